# baseline (speedup 1.0000x reference)
import functools

import jax
import jax.numpy as jnp
from jax.experimental import pallas as pl
from jax.experimental.pallas import tpu as pltpu

T = 24000
B = 4
CIN = 64
TB = 4096
HALO = 128
NT = -(-T // TB)


def _shift_r(a):
    return jnp.concatenate([a[:, :1], a[:, :-1]], axis=1)


def _shift_l(a):
    return jnp.concatenate([a[:, 1:], a[:, -1:]], axis=1)


def _lrelu(a):
    return jnp.where(a > 0, a, 0.2 * a)


def _conv3(ab, w, b):
    acat = jnp.concatenate([_shift_r(ab), ab, _shift_l(ab)], axis=0)
    return jnp.dot(w, acat, preferred_element_type=jnp.float32) + b


def _net_kernel(xl_ref, xm_ref, xr_ref, a1_ref, b1_ref, a2_ref, b2_ref,
                a3_ref, b3_ref, h1_ref, bc_ref, h2_ref, bo_ref, out_ref):
    i = pl.program_id(0)
    bf16 = jnp.bfloat16
    x = jnp.concatenate([xl_ref[...], xm_ref[...], xr_ref[...]],
                        axis=1).astype(bf16)
    t = TB * i - HALO + jax.lax.broadcasted_iota(jnp.int32, (1, TB + 2 * HALO), 1)
    valid = jnp.logical_and(t >= 0, t < T)
    x = jnp.where(valid, x, bf16(0))

    def act(y, mask=True):
        yb = y.astype(bf16)
        yb = _lrelu(yb)
        return jnp.where(valid, yb, bf16(0)) if mask else yb

    h = act(_conv3(x, a1_ref[...], b1_ref[...]))
    h = act(_conv3(h, a2_ref[...], b2_ref[...]))
    h = act(_conv3(h, a3_ref[...], b3_ref[...]))
    c = act(_conv3(h, h1_ref[...], bc_ref[...]), mask=False)[:, HALO:HALO + TB]
    out_ref[...] = jnp.dot(h2_ref[...], c,
                           preferred_element_type=jnp.float32) + bo_ref[...]


def _taps(w, groups):
    o, ig, _ = w.shape
    og = o // groups
    wg = jnp.transpose(w, (2, 0, 1)).reshape(3, groups, og, ig)
    eye = jnp.eye(groups, dtype=w.dtype)
    return jnp.einsum('tgoi,gh->tgohi', wg, eye).reshape(3, o, groups * ig)


def _batch_fold(m):
    _, mm, kk = m.shape
    eye = jnp.eye(B, dtype=m.dtype)
    folded = jnp.einsum('tmk,bc->bmtck', m, eye)
    return folded.reshape(B * mm, 3 * B * kk)


def _pointwise_block_diag(mats):
    rows = sum(m.shape[0] for m in mats)
    cols = sum(m.shape[1] for m in mats)
    out = jnp.zeros((rows, cols), dtype=mats[0].dtype)
    r = c = 0
    for m in mats:
        out = jax.lax.dynamic_update_slice(out, m, (r, c))
        r += m.shape[0]
        c += m.shape[1]
    return out


@jax.jit
def kernel(x, fb_w1, fb_b1, fb_w2, fb_b2, fb_w3, fb_b3,
           mod_w1, mod_b1, mod_w2, mod_b2,
           tm_w1, tm_b1, tm_w2, tm_b2,
           me_w1, me_b1, me_w2, me_b2,
           am_w1, am_b1, am_w2, am_b2,
           rt_w1, rt_b1, rt_w2, rt_b2):
    f32 = jnp.float32
    x2d = x.reshape(B * CIN, T)

    a1 = _batch_fold(_taps(fb_w1, 1))
    a2 = _batch_fold(_taps(fb_w2, 4))
    a3 = _batch_fold(_taps(fb_w3, 1))
    b1 = jnp.tile(fb_b1, B).reshape(-1, 1)
    b2 = jnp.tile(fb_b2, B).reshape(-1, 1)
    b3 = jnp.tile(fb_b3, B).reshape(-1, 1)

    head1 = jnp.concatenate([
        _taps(mod_w1, 4), _taps(tm_w1, 4), _taps(me_w1, 4),
        _taps(am_w1, 4), _taps(rt_w1, 1)], axis=1)
    h1m = _batch_fold(head1)
    bc = jnp.tile(jnp.concatenate([mod_b1, tm_b1, me_b1, am_b1, rt_b1]),
                  B).reshape(-1, 1)

    h2one = _pointwise_block_diag([
        mod_w2[:, :, 0], tm_w2[:, :, 0], me_w2[:, :, 0],
        am_w2[:, :, 0], rt_w2[:, :, 0]])
    h2m = jnp.kron(jnp.eye(B, dtype=h2one.dtype), h2one)
    bo = jnp.tile(jnp.concatenate([mod_b2, tm_b2, me_b2, am_b2, rt_b2]),
                  B).reshape(-1, 1)

    bf16 = jnp.bfloat16
    a1, a2, a3, h1m, h2m = (m.astype(bf16) for m in (a1, a2, a3, h1m, h2m))

    nhb = -(-T // HALO) - 1

    def full(shape):
        return pl.BlockSpec(shape, lambda i: (0,) * len(shape))

    out2d = pl.pallas_call(
        _net_kernel,
        grid=(NT,),
        in_specs=[
            pl.BlockSpec((B * CIN, HALO),
                         lambda i: (0, jnp.maximum(i * (TB // HALO) - 1, 0))),
            pl.BlockSpec((B * CIN, TB), lambda i: (0, i)),
            pl.BlockSpec((B * CIN, HALO),
                         lambda i: (0, jnp.minimum(i * (TB // HALO) + TB // HALO, nhb))),
            full((128, 768)), full((128, 1)),
            full((128, 384)), full((128, 1)),
            full((128, 384)), full((128, 1)),
            full((320, 384)), full((320, 1)),
            full((240, 320)), full((240, 1)),
        ],
        out_specs=pl.BlockSpec((240, TB), lambda i: (0, i)),
        out_shape=jax.ShapeDtypeStruct((240, T), f32),
        compiler_params=pltpu.CompilerParams(
            dimension_semantics=("parallel",)),
    )(x2d, x2d, x2d, a1, b1, a2, b2, a3, b3, h1m, bc, h2m, bo)

    return out2d.reshape(B, 60, T)


# speedup vs baseline: 1.3074x; 1.3074x over previous
import functools

import jax
import jax.numpy as jnp
from jax.experimental import pallas as pl
from jax.experimental.pallas import tpu as pltpu

T = 24000
B = 4
CIN = 64
TB = 3072
HALO = 128
NT = -(-T // TB)


def _shift_r(a):
    return jnp.concatenate([a[:, :1], a[:, :-1]], axis=1)


def _shift_l(a):
    return jnp.concatenate([a[:, 1:], a[:, -1:]], axis=1)


def _lrelu(a):
    return jnp.where(a > 0, a, 0.2 * a)


def _conv3(ab, w, b):
    acat = jnp.concatenate([_shift_r(ab), ab, _shift_l(ab)], axis=0)
    return jnp.dot(w, acat, preferred_element_type=jnp.float32) + b


def _net_kernel(xl_ref, xm_ref, xr_ref, a1_ref, b1_ref, a2_ref, b2_ref,
                a3_ref, b3_ref, h1_ref, bc_ref, h2_ref, bo_ref, out_ref):
    i = pl.program_id(0)
    bf16 = jnp.bfloat16
    x = jnp.concatenate([xl_ref[...], xm_ref[...], xr_ref[...]],
                        axis=1).astype(bf16)
    t = TB * i - HALO + jax.lax.broadcasted_iota(jnp.int32, (1, TB + 2 * HALO), 1)
    valid = jnp.logical_and(t >= 0, t < T)
    x = jnp.where(valid, x, bf16(0))

    def act(y, mask=True):
        yb = y.astype(bf16)
        yb = _lrelu(yb)
        return jnp.where(valid, yb, bf16(0)) if mask else yb

    h = act(_conv3(x, a1_ref[...], b1_ref[...]))
    h = act(_conv3(h, a2_ref[...], b2_ref[...]))
    h = act(_conv3(h, a3_ref[...], b3_ref[...]))
    c = act(_conv3(h, h1_ref[...], bc_ref[...]), mask=False)[:, HALO:HALO + TB]
    h2 = h2_ref[...]
    bo = bo_ref[...]
    for b in range(B):
        out_ref[b] = jnp.dot(h2, c[80 * b:80 * (b + 1)],
                             preferred_element_type=jnp.float32) + bo


def _taps(w, groups):
    o, ig, _ = w.shape
    og = o // groups
    wg = jnp.transpose(w, (2, 0, 1)).reshape(3, groups, og, ig)
    eye = jnp.eye(groups, dtype=w.dtype)
    return jnp.einsum('tgoi,gh->tgohi', wg, eye).reshape(3, o, groups * ig)


def _batch_fold(m):
    _, mm, kk = m.shape
    eye = jnp.eye(B, dtype=m.dtype)
    folded = jnp.einsum('tmk,bc->bmtck', m, eye)
    return folded.reshape(B * mm, 3 * B * kk)


def _pointwise_block_diag(mats):
    rows = sum(m.shape[0] for m in mats)
    cols = sum(m.shape[1] for m in mats)
    out = jnp.zeros((rows, cols), dtype=mats[0].dtype)
    r = c = 0
    for m in mats:
        out = jax.lax.dynamic_update_slice(out, m, (r, c))
        r += m.shape[0]
        c += m.shape[1]
    return out


@jax.jit
def kernel(x, fb_w1, fb_b1, fb_w2, fb_b2, fb_w3, fb_b3,
           mod_w1, mod_b1, mod_w2, mod_b2,
           tm_w1, tm_b1, tm_w2, tm_b2,
           me_w1, me_b1, me_w2, me_b2,
           am_w1, am_b1, am_w2, am_b2,
           rt_w1, rt_b1, rt_w2, rt_b2):
    f32 = jnp.float32
    x2d = x.reshape(B * CIN, T)

    a1 = _batch_fold(_taps(fb_w1, 1))
    a2 = _batch_fold(_taps(fb_w2, 4))
    a3 = _batch_fold(_taps(fb_w3, 1))
    b1 = jnp.tile(fb_b1, B).reshape(-1, 1)
    b2 = jnp.tile(fb_b2, B).reshape(-1, 1)
    b3 = jnp.tile(fb_b3, B).reshape(-1, 1)

    head1 = jnp.concatenate([
        _taps(mod_w1, 4), _taps(tm_w1, 4), _taps(me_w1, 4),
        _taps(am_w1, 4), _taps(rt_w1, 1)], axis=1)
    h1m = _batch_fold(head1)
    bc = jnp.tile(jnp.concatenate([mod_b1, tm_b1, me_b1, am_b1, rt_b1]),
                  B).reshape(-1, 1)

    h2m = _pointwise_block_diag([
        mod_w2[:, :, 0], tm_w2[:, :, 0], me_w2[:, :, 0],
        am_w2[:, :, 0], rt_w2[:, :, 0]])
    bo = jnp.concatenate([mod_b2, tm_b2, me_b2, am_b2, rt_b2]).reshape(-1, 1)

    bf16 = jnp.bfloat16
    a1, a2, a3, h1m, h2m = (m.astype(bf16) for m in (a1, a2, a3, h1m, h2m))

    nhb = -(-T // HALO) - 1

    def full(shape):
        return pl.BlockSpec(shape, lambda i: (0,) * len(shape))

    out2d = pl.pallas_call(
        _net_kernel,
        grid=(NT,),
        in_specs=[
            pl.BlockSpec((B * CIN, HALO),
                         lambda i: (0, jnp.maximum(i * (TB // HALO) - 1, 0))),
            pl.BlockSpec((B * CIN, TB), lambda i: (0, i)),
            pl.BlockSpec((B * CIN, HALO),
                         lambda i: (0, jnp.minimum(i * (TB // HALO) + TB // HALO, nhb))),
            full((128, 768)), full((128, 1)),
            full((128, 384)), full((128, 1)),
            full((128, 384)), full((128, 1)),
            full((320, 384)), full((320, 1)),
            full((60, 80)), full((60, 1)),
        ],
        out_specs=pl.BlockSpec((B, 60, TB), lambda i: (0, 0, i)),
        out_shape=jax.ShapeDtypeStruct((B, 60, T), f32),
        compiler_params=pltpu.CompilerParams(
            dimension_semantics=("parallel",)),
    )(x2d, x2d, x2d, a1, b1, a2, b2, a3, b3, h1m, bc, h2m, bo)

    return out2d


# speedup vs baseline: 1.3315x; 1.0184x over previous
import functools

import jax
import jax.numpy as jnp
from jax.experimental import pallas as pl
from jax.experimental.pallas import tpu as pltpu

T = 24000
B = 4
CIN = 64
TB = 4096
HALO = 128
NT = -(-T // TB)


def _shift_r(a):
    return jnp.concatenate([a[:, :1], a[:, :-1]], axis=1)


def _shift_l(a):
    return jnp.concatenate([a[:, 1:], a[:, -1:]], axis=1)


def _lrelu(a):
    return jnp.where(a > 0, a, 0.2 * a)


def _conv3(ab, w, b):
    acat = jnp.concatenate([_shift_r(ab), ab, _shift_l(ab)], axis=0)
    return jnp.dot(w, acat, preferred_element_type=jnp.float32) + b


def _net_kernel(xl_ref, xm_ref, xr_ref, a1_ref, b1_ref, a2_ref, b2_ref,
                a3_ref, b3_ref, h1_ref, bc_ref, h2_ref, bo_ref, out_ref):
    i = pl.program_id(0)
    bf16 = jnp.bfloat16
    x = jnp.concatenate([xl_ref[...], xm_ref[...], xr_ref[...]],
                        axis=1).astype(bf16)
    t = TB * i - HALO + jax.lax.broadcasted_iota(jnp.int32, (1, TB + 2 * HALO), 1)
    valid = jnp.logical_and(t >= 0, t < T)
    x = jnp.where(valid, x, bf16(0))

    def act(y, mask=True):
        yb = y.astype(bf16)
        yb = _lrelu(yb)
        return jnp.where(valid, yb, bf16(0)) if mask else yb

    h = act(_conv3(x, a1_ref[...], b1_ref[...]))
    h = act(_conv3(h, a2_ref[...], b2_ref[...]))
    h = act(_conv3(h, a3_ref[...], b3_ref[...]))
    c = act(_conv3(h, h1_ref[...], bc_ref[...]), mask=False)[:, HALO:HALO + TB]
    h2 = h2_ref[...]
    bo = bo_ref[...]
    for b in range(B):
        out_ref[b] = jnp.dot(h2, c[80 * b:80 * (b + 1)],
                             preferred_element_type=jnp.float32) + bo


def _taps(w, groups):
    o, ig, _ = w.shape
    og = o // groups
    wg = jnp.transpose(w, (2, 0, 1)).reshape(3, groups, og, ig)
    eye = jnp.eye(groups, dtype=w.dtype)
    return jnp.einsum('tgoi,gh->tgohi', wg, eye).reshape(3, o, groups * ig)


def _batch_fold(m):
    _, mm, kk = m.shape
    eye = jnp.eye(B, dtype=m.dtype)
    folded = jnp.einsum('tmk,bc->bmtck', m, eye)
    return folded.reshape(B * mm, 3 * B * kk)


def _pointwise_block_diag(mats):
    rows = sum(m.shape[0] for m in mats)
    cols = sum(m.shape[1] for m in mats)
    out = jnp.zeros((rows, cols), dtype=mats[0].dtype)
    r = c = 0
    for m in mats:
        out = jax.lax.dynamic_update_slice(out, m, (r, c))
        r += m.shape[0]
        c += m.shape[1]
    return out


@jax.jit
def kernel(x, fb_w1, fb_b1, fb_w2, fb_b2, fb_w3, fb_b3,
           mod_w1, mod_b1, mod_w2, mod_b2,
           tm_w1, tm_b1, tm_w2, tm_b2,
           me_w1, me_b1, me_w2, me_b2,
           am_w1, am_b1, am_w2, am_b2,
           rt_w1, rt_b1, rt_w2, rt_b2):
    f32 = jnp.float32
    x2d = x.reshape(B * CIN, T)

    a1 = _batch_fold(_taps(fb_w1, 1))
    a2 = _batch_fold(_taps(fb_w2, 4))
    a3 = _batch_fold(_taps(fb_w3, 1))
    b1 = jnp.tile(fb_b1, B).reshape(-1, 1)
    b2 = jnp.tile(fb_b2, B).reshape(-1, 1)
    b3 = jnp.tile(fb_b3, B).reshape(-1, 1)

    head1 = jnp.concatenate([
        _taps(mod_w1, 4), _taps(tm_w1, 4), _taps(me_w1, 4),
        _taps(am_w1, 4), _taps(rt_w1, 1)], axis=1)
    h1m = _batch_fold(head1)
    bc = jnp.tile(jnp.concatenate([mod_b1, tm_b1, me_b1, am_b1, rt_b1]),
                  B).reshape(-1, 1)

    h2m = _pointwise_block_diag([
        mod_w2[:, :, 0], tm_w2[:, :, 0], me_w2[:, :, 0],
        am_w2[:, :, 0], rt_w2[:, :, 0]])
    bo = jnp.concatenate([mod_b2, tm_b2, me_b2, am_b2, rt_b2]).reshape(-1, 1)

    bf16 = jnp.bfloat16
    a1, a2, a3, h1m, h2m = (m.astype(bf16) for m in (a1, a2, a3, h1m, h2m))

    nhb = -(-T // HALO) - 1

    def full(shape):
        return pl.BlockSpec(shape, lambda i: (0,) * len(shape))

    out2d = pl.pallas_call(
        _net_kernel,
        grid=(NT,),
        in_specs=[
            pl.BlockSpec((B * CIN, HALO),
                         lambda i: (0, jnp.maximum(i * (TB // HALO) - 1, 0))),
            pl.BlockSpec((B * CIN, TB), lambda i: (0, i)),
            pl.BlockSpec((B * CIN, HALO),
                         lambda i: (0, jnp.minimum(i * (TB // HALO) + TB // HALO, nhb))),
            full((128, 768)), full((128, 1)),
            full((128, 384)), full((128, 1)),
            full((128, 384)), full((128, 1)),
            full((320, 384)), full((320, 1)),
            full((60, 80)), full((60, 1)),
        ],
        out_specs=pl.BlockSpec((B, 60, TB), lambda i: (0, 0, i)),
        out_shape=jax.ShapeDtypeStruct((B, 60, T), f32),
        compiler_params=pltpu.CompilerParams(
            dimension_semantics=("parallel",)),
    )(x2d, x2d, x2d, a1, b1, a2, b2, a3, b3, h1m, bc, h2m, bo)

    return out2d
